# Initial kernel scaffold; baseline (speedup 1.0000x reference)
import jax
import jax.numpy as jnp
from jax.experimental import pallas as pl


def kernel(x, table):
    raise NotImplementedError("write your pallas kernel here")


# speedup vs baseline: 1.5578x; 1.5578x over previous
import functools

import jax
import jax.numpy as jnp
from jax import lax
from jax.experimental import pallas as pl
from jax.experimental.pallas import tpu as pltpu
from jax.experimental.pallas import tpu_sc as plsc

D = 32
CHUNK = 128
K = 8
NC, NS = 2, 16
NW = NC * NS
GROUP = K * CHUNK


@functools.lru_cache(maxsize=None)
def _make_gather(B: int, V: int):
  assert B % (NW * GROUP) == 0
  b_per_w = B // NW
  n_groups = b_per_w // GROUP
  mesh = plsc.VectorSubcoreMesh(core_axis_name="c", subcore_axis_name="s")

  @functools.partial(
      pl.kernel,
      out_type=jax.ShapeDtypeStruct((B, D), jnp.float32),
      mesh=mesh,
      compiler_params=pltpu.CompilerParams(use_tc_tiling_on_sc=False),
      scratch_types=[
          pltpu.VMEM((b_per_w,), jnp.int32),
          pltpu.VMEM((GROUP, D), jnp.float32),
          pltpu.SemaphoreType.DMA,
          pltpu.SemaphoreType.DMA,
      ],
  )
  def k(idx_hbm, table_hbm, out_hbm, idx_v, rows_v, gsem, psem):
    wid = lax.axis_index("s") * NC + lax.axis_index("c")
    base = wid * b_per_w
    pltpu.sync_copy(idx_hbm.at[pl.ds(base, b_per_w)], idx_v)

    def group_body(g, _):
      gbase = g * GROUP
      copies = [
          pltpu.async_copy(
              table_hbm.at[idx_v.at[pl.ds(gbase + b * CHUNK, CHUNK)]],
              rows_v.at[pl.ds(b * CHUNK, CHUNK)],
              gsem,
          )
          for b in range(K)
      ]
      for c in copies:
        c.wait()
      pltpu.async_copy(
          rows_v, out_hbm.at[pl.ds(base + gbase, GROUP)], psem
      ).wait()
      return 0

    lax.fori_loop(0, n_groups, group_body, 0)

  return k


def kernel(x, table):
  Bt, F = x.shape
  V, d = table.shape
  assert d == D
  idx = x.reshape(Bt * F).astype(jnp.int32)
  out = _make_gather(Bt * F, V)(idx, table)
  return out.reshape(Bt, F, D)
